# baseline (speedup 1.0000x reference)
import functools

import jax
import jax.numpy as jnp
from jax import lax
from jax.experimental import pallas as pl
from jax.experimental.pallas import tpu as pltpu
from jax.experimental.pallas import tpu_sc as plsc

B = 16384
F = 26
L = 20
V = 100000
D = 16
ND = 13
C = 16
KEEP_PROB = 0.8
TEMPERATURE = 0.1

NC = 2
NS = 16
NW = NC * NS
BAGS = B * F
PW = BAGS // NW
CB = 128
NCHUNK = PW // CB
RPC = CB * L
NIDX = RPC // 128


BPW = B // NW
SUBS = BPW // CB


def _sc_embedding_bag(idx4d, tables):
    mesh = plsc.VectorSubcoreMesh(core_axis_name="c", subcore_axis_name="s")

    @functools.partial(
        pl.kernel,
        mesh=mesh,
        compiler_params=pltpu.CompilerParams(use_tc_tiling_on_sc=False),
        out_type=jax.ShapeDtypeStruct((B, F, D), jnp.float32),
        scratch_types=[
            pltpu.VMEM((NIDX, 1, 128), jnp.int32),
            pltpu.VMEM((RPC, D), jnp.float32),
            pltpu.VMEM((CB, D), jnp.float32),
            pltpu.SemaphoreType.DMA,
        ],
    )
    def k(idx_hbm, tbl_hbm, out_hbm, idx_v, rows_v, out_v, sem):
        wid = lax.axis_index("s") * NC + lax.axis_index("c")

        def chunk_body(g, carry):
            f = g // SUBS
            sub = g - f * SUBS
            b0 = wid * BPW + sub * CB
            row0 = wid * (BPW * L // 128) + sub * NIDX
            pltpu.sync_copy(idx_hbm.at[f, pl.ds(row0, NIDX)], idx_v)
            copies = []
            for j in range(NIDX):
                copies.append(pltpu.async_copy(
                    tbl_hbm.at[f].at[idx_v.at[j, 0]],
                    rows_v.at[pl.ds(j * 128, 128)],
                    sem,
                ))
            for cp in copies:
                cp.wait()

            def pool_body(i, c2):
                e = i * L
                acc = rows_v[e]
                for l in range(1, L):
                    acc = acc + rows_v[e + l]
                out_v[i] = acc
                return c2
            lax.fori_loop(0, CB, pool_body, 0)
            pltpu.sync_copy(out_v, out_hbm.at[pl.ds(b0, CB), f])
            return carry

        lax.fori_loop(0, F * SUBS, chunk_body, 0)

    return k(idx4d, tables)


BB = 2048


def _autodis_body(x_ref, w_ref, pm_ref, me_ref, ex_ref, o_ref):
    x = x_ref[...]
    xe = jnp.dot(x, ex_ref[...], preferred_element_type=jnp.float32)
    pre = xe * w_ref[...]
    h = jnp.where(pre >= 0, pre, 0.01 * pre)
    xb = jnp.dot(h, pm_ref[...], precision=lax.Precision.HIGHEST,
                 preferred_element_type=jnp.float32) + KEEP_PROB * h
    parts = []
    for n in range(ND):
        g = xb[:, n * C:(n + 1) * C] * (1.0 / TEMPERATURE)
        m = jnp.max(g, axis=1, keepdims=True)
        e = jnp.exp(g - m)
        parts.append(e / jnp.sum(e, axis=1, keepdims=True))
    xh = jnp.concatenate(parts, axis=1)
    o_ref[...] = jnp.dot(xh, me_ref[...], precision=lax.Precision.HIGHEST,
                         preferred_element_type=jnp.float32)


def _tc_autodis(dense_input, w_row, pm_bd, me_bd, expand):
    grid = (B // BB,)
    return pl.pallas_call(
        _autodis_body,
        grid=grid,
        in_specs=[
            pl.BlockSpec((BB, ND), lambda i: (i, 0)),
            pl.BlockSpec((1, ND * C), lambda i: (0, 0)),
            pl.BlockSpec((ND * C, ND * C), lambda i: (0, 0)),
            pl.BlockSpec((ND * C, ND * C), lambda i: (0, 0)),
            pl.BlockSpec((ND, ND * C), lambda i: (0, 0)),
        ],
        out_specs=pl.BlockSpec((BB, ND * C), lambda i: (i, 0)),
        out_shape=jax.ShapeDtypeStruct((B, ND * C), jnp.float32),
    )(dense_input, w_row, pm_bd, me_bd, expand)


def kernel(indices, dense_input, tables, meta_emb, proj_w, proj_m):
    idx32 = indices.astype(jnp.int32)
    idx4d = jnp.transpose(idx32, (1, 0, 2)).reshape(F, B * L // 128, 1, 128)

    eye = jnp.eye(ND, dtype=jnp.float32)
    pm_bd = jnp.einsum('mn,nij->mjni', eye, proj_m).reshape(ND * C, ND * C)
    me_bd = jnp.einsum('mn,ncd->mcnd', eye, meta_emb).reshape(ND * C, ND * C)
    w_row = proj_w.reshape(1, ND * C)
    expand = jnp.repeat(eye, C, axis=1)

    pooled = _sc_embedding_bag(idx4d, tables)
    dense_out = _tc_autodis(dense_input, w_row, pm_bd, me_bd, expand)

    sparse_out = pooled.reshape(B, F * D)
    return jnp.concatenate([sparse_out, dense_out], axis=1)


# speedup vs baseline: 1.3472x; 1.3472x over previous
import functools

import jax
import jax.numpy as jnp
from jax import lax
from jax.experimental import pallas as pl
from jax.experimental.pallas import tpu as pltpu
from jax.experimental.pallas import tpu_sc as plsc

B = 16384
F = 26
L = 20
V = 100000
D = 16
ND = 13
C = 16
KEEP_PROB = 0.8
TEMPERATURE = 0.1

NC = 2
NS = 16
NW = NC * NS
BAGS = B * F
CB = 128
RPC = CB * L
NIDX = L
BPW = B // NW
SUBS = BPW // CB


def _repack_body(t2_ref, o_ref):
    x = t2_ref[0]
    xt = jnp.swapaxes(x, 0, 1)
    o_ref[0] = xt.reshape(V // 8, 8, D).reshape(V // 8, 8 * D)


def _repack_tables(t2):
    return pl.pallas_call(
        _repack_body,
        grid=(F,),
        in_specs=[pl.BlockSpec((1, D, V), lambda f: (f, 0, 0))],
        out_specs=pl.BlockSpec((1, V * D // 128, 128), lambda f: (f, 0, 0)),
        out_shape=jax.ShapeDtypeStruct((F, V * D // 128, 128), jnp.float32),
    )(t2)


def _sc_embedding_bag(idx3d, tables_rm):
    mesh = plsc.VectorSubcoreMesh(core_axis_name="c", subcore_axis_name="s")

    @functools.partial(
        pl.kernel,
        mesh=mesh,
        compiler_params=pltpu.CompilerParams(use_tc_tiling_on_sc=False),
        out_type=jax.ShapeDtypeStruct((B, F * D), jnp.float32),
        scratch_types=[
            pltpu.VMEM((NIDX, 128), jnp.int32),
            pltpu.VMEM((RPC, D), jnp.float32),
            pltpu.VMEM((CB, D), jnp.float32),
            pltpu.SemaphoreType.DMA,
        ],
    )
    def k(idx_hbm, tbl_hbm, out_hbm, idx_v, rows_v, out_v, sem):
        wid = lax.axis_index("s") * NC + lax.axis_index("c")

        def chunk_body(g, carry):
            f = g // SUBS
            sub = g - f * SUBS
            b0 = wid * BPW + sub * CB
            pltpu.sync_copy(idx_hbm.at[f, :, pl.ds(b0, CB)], idx_v)
            copies = []
            for j in range(NIDX):
                copies.append(pltpu.async_copy(
                    tbl_hbm.at[f].at[idx_v.at[j]],
                    rows_v.at[pl.ds(j * CB, CB)],
                    sem,
                ))
            for cp in copies:
                cp.wait()

            def pool_body(i, c2):
                acc = rows_v[i]
                for l in range(1, L):
                    acc = acc + rows_v[l * CB + i]
                out_v[i] = acc
                return c2
            lax.fori_loop(0, CB, pool_body, 0)
            pltpu.sync_copy(out_v, out_hbm.at[pl.ds(b0, CB),
                                              pl.ds(f * D, D)])
            return carry

        lax.fori_loop(0, F * SUBS, chunk_body, 0)

    return k(idx3d, tables_rm)


BB = 2048
HI = lax.Precision.HIGHEST


def _autodis_body(x_ref, w_ref, pm_ref, me_ref, ex_ref, bd_ref, fm_ref,
                  o_ref):
    x = x_ref[...]
    xe = jnp.dot(x, ex_ref[...], precision=HI,
                 preferred_element_type=jnp.float32)
    pre = xe * w_ref[...]
    h = jnp.where(pre >= 0, pre, 0.01 * pre)
    xb = jnp.dot(h, pm_ref[...], precision=HI,
                 preferred_element_type=jnp.float32) + KEEP_PROB * h
    g = xb * (1.0 / TEMPERATURE)
    r = jnp.maximum(g, jnp.roll(g, -1, axis=1))
    r = jnp.maximum(r, jnp.roll(r, -2, axis=1))
    r = jnp.maximum(r, jnp.roll(r, -4, axis=1))
    r = jnp.maximum(r, jnp.roll(r, -8, axis=1))
    m = jnp.dot(r * fm_ref[...], bd_ref[...], precision=HI,
                preferred_element_type=jnp.float32)
    e = jnp.exp(g - m)
    s = jnp.dot(e, bd_ref[...], precision=HI,
                preferred_element_type=jnp.float32)
    o_ref[...] = jnp.dot(e / s, me_ref[...], precision=HI,
                         preferred_element_type=jnp.float32)


def _tc_autodis(dense_input, w_row, pm_bd, me_bd, expand, ones_bd, fmask):
    return pl.pallas_call(
        _autodis_body,
        grid=(B // BB,),
        in_specs=[
            pl.BlockSpec((BB, ND), lambda i: (i, 0)),
            pl.BlockSpec((1, ND * C), lambda i: (0, 0)),
            pl.BlockSpec((ND * C, ND * C), lambda i: (0, 0)),
            pl.BlockSpec((ND * C, ND * C), lambda i: (0, 0)),
            pl.BlockSpec((ND, ND * C), lambda i: (0, 0)),
            pl.BlockSpec((ND * C, ND * C), lambda i: (0, 0)),
            pl.BlockSpec((1, ND * C), lambda i: (0, 0)),
        ],
        out_specs=pl.BlockSpec((BB, ND * C), lambda i: (i, 0)),
        out_shape=jax.ShapeDtypeStruct((B, ND * C), jnp.float32),
    )(dense_input, w_row, pm_bd, me_bd, expand, ones_bd, fmask)


def kernel(indices, dense_input, tables, meta_emb, proj_w, proj_m):
    idx3d = jnp.transpose(indices.astype(jnp.int32), (1, 2, 0))

    eye = jnp.eye(ND, dtype=jnp.float32)
    pm_bd = jnp.einsum('mn,nij->mjni', eye, proj_m).reshape(ND * C, ND * C)
    me_bd = jnp.einsum('mn,ncd->mcnd', eye, meta_emb).reshape(ND * C, ND * C)
    w_row = proj_w.reshape(1, ND * C)
    expand = jnp.repeat(eye, C, axis=1)
    ones_bd = jnp.einsum('mn,c,d->mcnd', eye,
                         jnp.ones((C,), jnp.float32),
                         jnp.ones((C,), jnp.float32)).reshape(ND * C, ND * C)
    fmask = (jnp.arange(ND * C, dtype=jnp.int32) % C == 0
             ).astype(jnp.float32).reshape(1, ND * C)

    sparse_out = _sc_embedding_bag(idx3d, tables)
    dense_out = _tc_autodis(dense_input, w_row, pm_bd, me_bd, expand,
                            ones_bd, fmask)

    return jnp.concatenate([sparse_out, dense_out], axis=1)


# speedup vs baseline: 2.2250x; 1.6516x over previous
import functools

import jax
import jax.numpy as jnp
from jax import lax
from jax.experimental import pallas as pl
from jax.experimental.pallas import tpu as pltpu
from jax.experimental.pallas import tpu_sc as plsc

B = 16384
F = 26
L = 20
V = 100000
D = 16
ND = 13
C = 16
KEEP_PROB = 0.8
TEMPERATURE = 0.1

NC = 2
NS = 16
NW = NC * NS
BAGS = B * F
CB = 128
RPC = CB * L
NIDX = L
BPW = B // NW
SUBS = BPW // CB


SEG = V // 8


def _repack_body(t2_ref, o_ref):
    x = t2_ref[0]
    yt = jnp.concatenate(
        [x[:, s * SEG:(s + 1) * SEG] for s in range(8)], axis=0)
    o_ref[0] = jnp.swapaxes(yt, 0, 1)


def _repack_tables(t2):
    return pl.pallas_call(
        _repack_body,
        grid=(F,),
        in_specs=[pl.BlockSpec((1, D, V), lambda f: (f, 0, 0))],
        out_specs=pl.BlockSpec((1, V * D // 128, 128), lambda f: (f, 0, 0)),
        out_shape=jax.ShapeDtypeStruct((F, V * D // 128, 128), jnp.float32),
    )(t2)


def _sc_embedding_bag(idx3d, tables_rm):
    mesh = plsc.VectorSubcoreMesh(core_axis_name="c", subcore_axis_name="s")

    @functools.partial(
        pl.kernel,
        mesh=mesh,
        compiler_params=pltpu.CompilerParams(use_tc_tiling_on_sc=False),
        out_type=jax.ShapeDtypeStruct((B, F * D), jnp.float32),
        scratch_types=[
            pltpu.VMEM((NIDX, 128), jnp.int32),
            pltpu.VMEM((RPC, D), jnp.float32),
            pltpu.VMEM((CB, D), jnp.float32),
            pltpu.SemaphoreType.DMA,
        ],
    )
    def k(idx_hbm, tbl_hbm, out_hbm, idx_v, rows_v, out_v, sem):
        wid = lax.axis_index("s") * NC + lax.axis_index("c")

        def chunk_body(g, carry):
            f = g // SUBS
            sub = g - f * SUBS
            b0 = wid * BPW + sub * CB
            pltpu.sync_copy(idx_hbm.at[f, :, pl.ds(b0, CB)], idx_v)
            copies = []
            for j in range(NIDX):
                copies.append(pltpu.async_copy(
                    tbl_hbm.at[f].at[idx_v.at[j]],
                    rows_v.at[pl.ds(j * CB, CB)],
                    sem,
                ))
            for cp in copies:
                cp.wait()

            def pool_body(i, c2):
                acc = rows_v[i]
                for l in range(1, L):
                    acc = acc + rows_v[l * CB + i]
                out_v[i] = acc
                return c2
            lax.fori_loop(0, CB, pool_body, 0)
            pltpu.sync_copy(out_v, out_hbm.at[pl.ds(b0, CB),
                                              pl.ds(f * D, D)])
            return carry

        lax.fori_loop(0, F * SUBS, chunk_body, 0)

    return k(idx3d, tables_rm)


BB = 2048
HI = lax.Precision.HIGHEST


def _autodis_body(x_ref, w_ref, pm_ref, me_ref, ex_ref, bd_ref, fm_ref,
                  o_ref):
    x = x_ref[...]
    xe = jnp.dot(x, ex_ref[...], precision=HI,
                 preferred_element_type=jnp.float32)
    pre = xe * w_ref[...]
    h = jnp.where(pre >= 0, pre, 0.01 * pre)
    xb = jnp.dot(h, pm_ref[...], precision=HI,
                 preferred_element_type=jnp.float32) + KEEP_PROB * h
    g = xb * (1.0 / TEMPERATURE)
    r = jnp.maximum(g, jnp.roll(g, -1, axis=1))
    r = jnp.maximum(r, jnp.roll(r, -2, axis=1))
    r = jnp.maximum(r, jnp.roll(r, -4, axis=1))
    r = jnp.maximum(r, jnp.roll(r, -8, axis=1))
    m = jnp.dot(r * fm_ref[...], bd_ref[...], precision=HI,
                preferred_element_type=jnp.float32)
    e = jnp.exp(g - m)
    s = jnp.dot(e, bd_ref[...], precision=HI,
                preferred_element_type=jnp.float32)
    o_ref[...] = jnp.dot(e / s, me_ref[...], precision=HI,
                         preferred_element_type=jnp.float32)


def _tc_autodis(dense_input, w_row, pm_bd, me_bd, expand, ones_bd, fmask):
    return pl.pallas_call(
        _autodis_body,
        grid=(B // BB,),
        in_specs=[
            pl.BlockSpec((BB, ND), lambda i: (i, 0)),
            pl.BlockSpec((1, ND * C), lambda i: (0, 0)),
            pl.BlockSpec((ND * C, ND * C), lambda i: (0, 0)),
            pl.BlockSpec((ND * C, ND * C), lambda i: (0, 0)),
            pl.BlockSpec((ND, ND * C), lambda i: (0, 0)),
            pl.BlockSpec((ND * C, ND * C), lambda i: (0, 0)),
            pl.BlockSpec((1, ND * C), lambda i: (0, 0)),
        ],
        out_specs=pl.BlockSpec((BB, ND * C), lambda i: (i, 0)),
        out_shape=jax.ShapeDtypeStruct((B, ND * C), jnp.float32),
    )(dense_input, w_row, pm_bd, me_bd, expand, ones_bd, fmask)


def kernel(indices, dense_input, tables, meta_emb, proj_w, proj_m):
    idx32 = indices.astype(jnp.int32)
    gidx = (idx32 % SEG) * 8 + idx32 // SEG
    idx3d = jnp.transpose(gidx, (1, 2, 0))
    t2 = jnp.transpose(tables, (0, 2, 1))
    tables_rm = _repack_tables(t2).reshape(F, V, D)

    eye = jnp.eye(ND, dtype=jnp.float32)
    pm_bd = jnp.einsum('mn,nij->mjni', eye, proj_m).reshape(ND * C, ND * C)
    me_bd = jnp.einsum('mn,ncd->mcnd', eye, meta_emb).reshape(ND * C, ND * C)
    w_row = proj_w.reshape(1, ND * C)
    expand = jnp.repeat(eye, C, axis=1)
    ones_bd = jnp.einsum('mn,c,d->mcnd', eye,
                         jnp.ones((C,), jnp.float32),
                         jnp.ones((C,), jnp.float32)).reshape(ND * C, ND * C)
    fmask = (jnp.arange(ND * C, dtype=jnp.int32) % C == 0
             ).astype(jnp.float32).reshape(1, ND * C)

    sparse_out = _sc_embedding_bag(idx3d, tables_rm)
    dense_out = _tc_autodis(dense_input, w_row, pm_bd, me_bd, expand,
                            ones_bd, fmask)

    return jnp.concatenate([sparse_out, dense_out], axis=1)


# speedup vs baseline: 3.0602x; 1.3754x over previous
import functools

import jax
import jax.numpy as jnp
from jax import lax
from jax.experimental import pallas as pl
from jax.experimental.pallas import tpu as pltpu
from jax.experimental.pallas import tpu_sc as plsc

B = 16384
F = 26
L = 20
V = 100000
D = 16
ND = 13
C = 16
KEEP_PROB = 0.8
TEMPERATURE = 0.1

NC = 2
NS = 16
NW = NC * NS
BAGS = B * F
CB = 128
RPC = CB * L
NIDX = L
BPW = B // NW
SUBS = BPW // CB


SEG = V // 8


def _repack_body(t2_ref, o_ref):
    x = t2_ref[0]
    yt = jnp.concatenate(
        [x[:, s * SEG:(s + 1) * SEG] for s in range(8)], axis=0)
    o_ref[0] = jnp.swapaxes(yt, 0, 1)


def _repack_tables(t2):
    return pl.pallas_call(
        _repack_body,
        grid=(F,),
        in_specs=[pl.BlockSpec((1, D, V), lambda f: (f, 0, 0))],
        out_specs=pl.BlockSpec((1, V * D // 128, 128), lambda f: (f, 0, 0)),
        out_shape=jax.ShapeDtypeStruct((F, V * D // 128, 128), jnp.float32),
    )(t2)


def _sc_embedding_bag(idx3d, tables_rm):
    mesh = plsc.VectorSubcoreMesh(core_axis_name="c", subcore_axis_name="s")

    @functools.partial(
        pl.kernel,
        mesh=mesh,
        compiler_params=pltpu.CompilerParams(use_tc_tiling_on_sc=False),
        out_type=jax.ShapeDtypeStruct((B, F * D), jnp.float32),
        scratch_types=[
            pltpu.VMEM((2, NIDX, 128), jnp.int32),
            pltpu.VMEM((2, RPC, D), jnp.float32),
            pltpu.VMEM((CB, D), jnp.float32),
            pltpu.SemaphoreType.DMA,
            pltpu.SemaphoreType.DMA,
        ],
    )
    def k(idx_hbm, tbl_hbm, out_hbm, idx_v, rows_v, out_v, semA, semB):
        wid = lax.axis_index("s") * NC + lax.axis_index("c")
        NCHUNK = F * SUBS
        sems = (semA, semB)

        def coords(g):
            f = g // SUBS
            sub = g - f * SUBS
            return f, wid * BPW + sub * CB

        def fire(g, buf):
            f, b0 = coords(g)
            pltpu.sync_copy(idx_hbm.at[f, :, pl.ds(b0, CB)], idx_v.at[buf])
            for j in range(NIDX):
                pltpu.async_copy(
                    tbl_hbm.at[f].at[idx_v.at[buf, j]],
                    rows_v.at[buf, pl.ds(j * CB, CB)],
                    sems[buf],
                )

        def drain_pool_store(g, buf):
            for j in range(NIDX):
                pltpu.make_async_copy(
                    tbl_hbm.at[0].at[pl.ds(0, CB)],
                    rows_v.at[buf, pl.ds(j * CB, CB)],
                    sems[buf],
                ).wait()
            rv = rows_v.at[buf]

            def pool_body(i, c2):
                acc = rv[i]
                for l in range(1, L):
                    acc = acc + rv[l * CB + i]
                out_v[i] = acc
                return c2
            lax.fori_loop(0, CB, pool_body, 0)
            f, b0 = coords(g)
            pltpu.sync_copy(out_v, out_hbm.at[pl.ds(b0, CB),
                                              pl.ds(f * D, D)])

        fire(0, 0)

        def super_body(s, carry):
            g0 = 2 * s
            fire(g0 + 1, 1)
            drain_pool_store(g0, 0)
            fire(jnp.minimum(g0 + 2, NCHUNK - 1), 0)
            drain_pool_store(g0 + 1, 1)
            return carry

        lax.fori_loop(0, NCHUNK // 2, super_body, 0)
        for j in range(NIDX):
            pltpu.make_async_copy(
                tbl_hbm.at[pl.ds(0, CB)],
                rows_v.at[0, pl.ds(j * CB, CB)],
                semA,
            ).wait()

    return k(idx3d, tables_rm)


BB = 2048
HI = lax.Precision.HIGHEST


def _autodis_body(x_ref, w_ref, pm_ref, me_ref, ex_ref, bd_ref, fm_ref,
                  o_ref):
    x = x_ref[...]
    xe = jnp.dot(x, ex_ref[...], precision=HI,
                 preferred_element_type=jnp.float32)
    pre = xe * w_ref[...]
    h = jnp.where(pre >= 0, pre, 0.01 * pre)
    xb = jnp.dot(h, pm_ref[...], precision=HI,
                 preferred_element_type=jnp.float32) + KEEP_PROB * h
    g = xb * (1.0 / TEMPERATURE)
    r = jnp.maximum(g, jnp.roll(g, -1, axis=1))
    r = jnp.maximum(r, jnp.roll(r, -2, axis=1))
    r = jnp.maximum(r, jnp.roll(r, -4, axis=1))
    r = jnp.maximum(r, jnp.roll(r, -8, axis=1))
    m = jnp.dot(r * fm_ref[...], bd_ref[...], precision=HI,
                preferred_element_type=jnp.float32)
    e = jnp.exp(g - m)
    s = jnp.dot(e, bd_ref[...], precision=HI,
                preferred_element_type=jnp.float32)
    o_ref[...] = jnp.dot(e / s, me_ref[...], precision=HI,
                         preferred_element_type=jnp.float32)


def _tc_autodis(dense_input, w_row, pm_bd, me_bd, expand, ones_bd, fmask):
    return pl.pallas_call(
        _autodis_body,
        grid=(B // BB,),
        in_specs=[
            pl.BlockSpec((BB, ND), lambda i: (i, 0)),
            pl.BlockSpec((1, ND * C), lambda i: (0, 0)),
            pl.BlockSpec((ND * C, ND * C), lambda i: (0, 0)),
            pl.BlockSpec((ND * C, ND * C), lambda i: (0, 0)),
            pl.BlockSpec((ND, ND * C), lambda i: (0, 0)),
            pl.BlockSpec((ND * C, ND * C), lambda i: (0, 0)),
            pl.BlockSpec((1, ND * C), lambda i: (0, 0)),
        ],
        out_specs=pl.BlockSpec((BB, ND * C), lambda i: (i, 0)),
        out_shape=jax.ShapeDtypeStruct((B, ND * C), jnp.float32),
    )(dense_input, w_row, pm_bd, me_bd, expand, ones_bd, fmask)


def kernel(indices, dense_input, tables, meta_emb, proj_w, proj_m):
    idx32 = indices.astype(jnp.int32)
    gidx = (idx32 % SEG) * 8 + idx32 // SEG
    idx3d = jnp.transpose(gidx, (1, 2, 0))
    t2 = jnp.transpose(tables, (0, 2, 1))
    tables_rm = _repack_tables(t2).reshape(F, V, D)

    eye = jnp.eye(ND, dtype=jnp.float32)
    pm_bd = jnp.einsum('mn,nij->mjni', eye, proj_m).reshape(ND * C, ND * C)
    me_bd = jnp.einsum('mn,ncd->mcnd', eye, meta_emb).reshape(ND * C, ND * C)
    w_row = proj_w.reshape(1, ND * C)
    expand = jnp.repeat(eye, C, axis=1)
    ones_bd = jnp.einsum('mn,c,d->mcnd', eye,
                         jnp.ones((C,), jnp.float32),
                         jnp.ones((C,), jnp.float32)).reshape(ND * C, ND * C)
    fmask = (jnp.arange(ND * C, dtype=jnp.int32) % C == 0
             ).astype(jnp.float32).reshape(1, ND * C)

    sparse_out = _sc_embedding_bag(idx3d, tables_rm)
    dense_out = _tc_autodis(dense_input, w_row, pm_bd, me_bd, expand,
                            ones_bd, fmask)

    return jnp.concatenate([sparse_out, dense_out], axis=1)
